# baseline (speedup 1.0000x reference)
import jax
import jax.numpy as jnp
from jax.experimental import pallas as pl
from jax.experimental.pallas import tpu as pltpu

H, CH, W = 128, 64, 256
NE2 = 8
NE3 = 64
O1 = 8
O2 = 12
HID = 32


def _leaky(x):
    return jnp.where(x > 0, x, 0.01 * x)


def _argmax0(a, n):
    mx = jnp.max(a, axis=0)
    iota = jax.lax.broadcasted_iota(jnp.int32, a.shape, 0)
    cand = jnp.where(a == mx[None, :], iota, n)
    return jnp.min(cand, axis=0).astype(jnp.int32)


def _routed(w_ref, b_ref, hm, ne, ci, co, dtype):
    wf = w_ref[...].reshape(ne * ci, co).astype(dtype)
    bf = b_ref[...].astype(dtype)
    w_aug = jnp.concatenate([wf, bf], axis=0)
    return jax.lax.dot_general(
        w_aug, hm, (((0,), (0,)), ((), ())),
        preferred_element_type=jnp.float32)


def _line_kernel(x_ref,
                 w10, b10, w11, b11, w12, b12,
                 w20, b20, w21, b21, w22, b22,
                 w30, b30, w31, b31, w32, b32,
                 out_ref):
    X = x_ref[0]

    h = _leaky(jnp.dot(w10[0], X, preferred_element_type=jnp.float32) + b10[0])
    h = _leaky(jnp.dot(w11[0], h, preferred_element_type=jnp.float32) + b11[0])
    s1 = jnp.dot(w12[0], h, preferred_element_type=jnp.float32) + b12[0]
    inds1 = _argmax0(s1, O1)

    e_iota2 = jax.lax.broadcasted_iota(jnp.int32, (NE2, 1, W), 0)
    m2 = (e_iota2 == inds1[None, None, :]).astype(jnp.float32)
    m2_2d = m2.reshape(NE2, W)

    hm = jnp.concatenate([(X[None] * m2).reshape(NE2 * CH, W), m2_2d], axis=0)
    h = _leaky(_routed(w20, b20, hm, NE2, CH, HID, jnp.float32))
    hm = jnp.concatenate([(h[None] * m2).reshape(NE2 * HID, W), m2_2d], axis=0)
    h = _leaky(_routed(w21, b21, hm, NE2, HID, HID, jnp.float32))
    hm = jnp.concatenate([(h[None] * m2).reshape(NE2 * HID, W), m2_2d], axis=0)
    s2 = _routed(w22, b22, hm, NE2, HID, O2, jnp.float32)

    inds2 = _argmax0(s2, O2)
    inds12_raw = inds1 * 8 + inds2 - 2
    inds12 = jnp.clip(inds12_raw, 0, NE3 - 1)

    e_iota3 = jax.lax.broadcasted_iota(jnp.int32, (NE3, 1, W), 0)
    m3 = (e_iota3 == inds12[None, None, :]).astype(jnp.bfloat16)
    m3_2d = m3.reshape(NE3, W)
    Xb = X.astype(jnp.bfloat16)

    hm = jnp.concatenate([(Xb[None] * m3).reshape(NE3 * CH, W), m3_2d], axis=0)
    h = _leaky(_routed(w30, b30, hm, NE3, CH, HID, jnp.bfloat16))
    hb = h.astype(jnp.bfloat16)
    hm = jnp.concatenate([(hb[None] * m3).reshape(NE3 * HID, W), m3_2d], axis=0)
    h = _leaky(_routed(w31, b31, hm, NE3, HID, HID, jnp.bfloat16))
    hb = h.astype(jnp.bfloat16)
    hm = jnp.concatenate([(hb[None] * m3).reshape(NE3 * HID, W), m3_2d], axis=0)
    s3 = _routed(w32, b32, hm, NE3, HID, O2, jnp.bfloat16)

    inds3 = _argmax0(s3, O2)
    out_ref[0, 0] = jnp.clip(inds12_raw * 8 + inds3 - 2, 0, 511)


def kernel(x_in, c1_w0, c1_b0, c1_w1, c1_b1, c1_w2, c1_b2,
           c2_w0, c2_b0, c2_w1, c2_b1, c2_w2, c2_b2,
           c3_w0, c3_b0, c3_w1, c3_b1, c3_w2, c3_b2):
    x_t = jnp.transpose(x_in[0], (1, 0, 2))

    def wspec(ne, i, o):
        return pl.BlockSpec((ne, i, o), lambda h: (h, 0, 0))

    def bspec(ne, o):
        return pl.BlockSpec((ne, o), lambda h: (h, 0))

    in_specs = [
        pl.BlockSpec((1, CH, W), lambda h: (h, 0, 0)),
        pl.BlockSpec((1, HID, CH), lambda h: (h, 0, 0)),
        pl.BlockSpec((1, HID, 1), lambda h: (h, 0, 0)),
        pl.BlockSpec((1, HID, HID), lambda h: (h, 0, 0)),
        pl.BlockSpec((1, HID, 1), lambda h: (h, 0, 0)),
        pl.BlockSpec((1, O1, HID), lambda h: (h, 0, 0)),
        pl.BlockSpec((1, O1, 1), lambda h: (h, 0, 0)),
        wspec(NE2, CH, HID), bspec(NE2, HID),
        wspec(NE2, HID, HID), bspec(NE2, HID),
        wspec(NE2, HID, O2), bspec(NE2, O2),
        wspec(NE3, CH, HID), bspec(NE3, HID),
        wspec(NE3, HID, HID), bspec(NE3, HID),
        wspec(NE3, HID, O2), bspec(NE3, O2),
    ]

    args = [
        x_t,
        c1_w0, c1_b0.reshape(H, HID, 1),
        c1_w1, c1_b1.reshape(H, HID, 1),
        c1_w2, c1_b2.reshape(H, O1, 1),
        c2_w0, c2_b0, c2_w1, c2_b1, c2_w2, c2_b2,
        c3_w0, c3_b0, c3_w1, c3_b1, c3_w2, c3_b2,
    ]

    out = pl.pallas_call(
        _line_kernel,
        grid=(H,),
        in_specs=in_specs,
        out_specs=pl.BlockSpec((1, 1, W), lambda h: (h, 0, 0)),
        out_shape=jax.ShapeDtypeStruct((H, 1, W), jnp.int32),
        compiler_params=pltpu.CompilerParams(
            dimension_semantics=("arbitrary",),
        ),
    )(*args)

    return out.reshape(1, 1, H, W)


# speedup vs baseline: 1.0381x; 1.0381x over previous
import jax
import jax.numpy as jnp
from jax.experimental import pallas as pl
from jax.experimental.pallas import tpu as pltpu

H, CH, W = 128, 64, 256
NE2 = 8
NE3 = 64
O1 = 8
O2 = 12
HID = 32
LPB = 2


def _leaky(x):
    return jnp.where(x > 0, x, 0.01 * x)


def _argmax0(a, n):
    mx = jnp.max(a, axis=0)
    iota = jax.lax.broadcasted_iota(jnp.int32, a.shape, 0)
    cand = jnp.where(a == mx[None, :], iota, n)
    return jnp.min(cand, axis=0).astype(jnp.int32)


def _routed(wf, bf, hm, dtype):
    w_aug = jnp.concatenate([wf.astype(dtype), bf.astype(dtype)], axis=0)
    return jax.lax.dot_general(
        w_aug, hm, (((0,), (0,)), ((), ())),
        preferred_element_type=jnp.float32)


def _one_line(X, ws):
    (w10, b10, w11, b11, w12, b12,
     w20, b20, w21, b21, w22, b22,
     w30, b30, w31, b31, w32, b32) = ws

    h = _leaky(jnp.dot(w10, X, preferred_element_type=jnp.float32) + b10)
    h = _leaky(jnp.dot(w11, h, preferred_element_type=jnp.float32) + b11)
    s1 = jnp.dot(w12, h, preferred_element_type=jnp.float32) + b12
    inds1 = _argmax0(s1, O1)

    e_iota2 = jax.lax.broadcasted_iota(jnp.int32, (NE2, 1, W), 0)
    m2 = (e_iota2 == inds1[None, None, :]).astype(jnp.float32)
    m2_2d = m2.reshape(NE2, W)

    hm = jnp.concatenate([(X[None] * m2).reshape(NE2 * CH, W), m2_2d], axis=0)
    h = _leaky(_routed(w20.reshape(NE2 * CH, HID), b20, hm, jnp.float32))
    hm = jnp.concatenate([(h[None] * m2).reshape(NE2 * HID, W), m2_2d], axis=0)
    h = _leaky(_routed(w21.reshape(NE2 * HID, HID), b21, hm, jnp.float32))
    hm = jnp.concatenate([(h[None] * m2).reshape(NE2 * HID, W), m2_2d], axis=0)
    s2 = _routed(w22.reshape(NE2 * HID, O2), b22, hm, jnp.float32)

    inds2 = _argmax0(s2, O2)
    inds12_raw = inds1 * 8 + inds2 - 2
    inds12 = jnp.clip(inds12_raw, 0, NE3 - 1)

    e_iota3 = jax.lax.broadcasted_iota(jnp.int32, (NE3, 1, W), 0)
    m3 = (e_iota3 == inds12[None, None, :]).astype(jnp.bfloat16)
    m3_2d = m3.reshape(NE3, W)
    Xb = X.astype(jnp.bfloat16)

    hm = jnp.concatenate([(Xb[None] * m3).reshape(NE3 * CH, W), m3_2d], axis=0)
    h = _leaky(_routed(w30.reshape(NE3 * CH, HID), b30, hm, jnp.bfloat16))
    hb = h.astype(jnp.bfloat16)
    hm = jnp.concatenate([(hb[None] * m3).reshape(NE3 * HID, W), m3_2d], axis=0)
    h = _leaky(_routed(w31.reshape(NE3 * HID, HID), b31, hm, jnp.bfloat16))
    hb = h.astype(jnp.bfloat16)
    hm = jnp.concatenate([(hb[None] * m3).reshape(NE3 * HID, W), m3_2d], axis=0)
    s3 = _routed(w32.reshape(NE3 * HID, O2), b32, hm, jnp.bfloat16)

    inds3 = _argmax0(s3, O2)
    return jnp.clip(inds12_raw * 8 + inds3 - 2, 0, 511)


def _line_kernel(x_ref,
                 w10, b10, w11, b11, w12, b12,
                 w20, b20, w21, b21, w22, b22,
                 w30, b30, w31, b31, w32, b32,
                 out_ref):
    for j in range(LPB):
        ws = (w10[j], b10[j], w11[j], b11[j], w12[j], b12[j],
              w20[j * NE2:(j + 1) * NE2], b20[j * NE2:(j + 1) * NE2],
              w21[j * NE2:(j + 1) * NE2], b21[j * NE2:(j + 1) * NE2],
              w22[j * NE2:(j + 1) * NE2], b22[j * NE2:(j + 1) * NE2],
              w30[j * NE3:(j + 1) * NE3], b30[j * NE3:(j + 1) * NE3],
              w31[j * NE3:(j + 1) * NE3], b31[j * NE3:(j + 1) * NE3],
              w32[j * NE3:(j + 1) * NE3], b32[j * NE3:(j + 1) * NE3])
        out_ref[j, 0] = _one_line(x_ref[j], ws)


def kernel(x_in, c1_w0, c1_b0, c1_w1, c1_b1, c1_w2, c1_b2,
           c2_w0, c2_b0, c2_w1, c2_b1, c2_w2, c2_b2,
           c3_w0, c3_b0, c3_w1, c3_b1, c3_w2, c3_b2):
    x_t = jnp.transpose(x_in[0], (1, 0, 2))

    def wspec(ne, i, o):
        return pl.BlockSpec((LPB * ne, i, o), lambda h: (h, 0, 0))

    def bspec(ne, o):
        return pl.BlockSpec((LPB * ne, o), lambda h: (h, 0))

    in_specs = [
        pl.BlockSpec((LPB, CH, W), lambda h: (h, 0, 0)),
        pl.BlockSpec((LPB, HID, CH), lambda h: (h, 0, 0)),
        pl.BlockSpec((LPB, HID, 1), lambda h: (h, 0, 0)),
        pl.BlockSpec((LPB, HID, HID), lambda h: (h, 0, 0)),
        pl.BlockSpec((LPB, HID, 1), lambda h: (h, 0, 0)),
        pl.BlockSpec((LPB, O1, HID), lambda h: (h, 0, 0)),
        pl.BlockSpec((LPB, O1, 1), lambda h: (h, 0, 0)),
        wspec(NE2, CH, HID), bspec(NE2, HID),
        wspec(NE2, HID, HID), bspec(NE2, HID),
        wspec(NE2, HID, O2), bspec(NE2, O2),
        wspec(NE3, CH, HID), bspec(NE3, HID),
        wspec(NE3, HID, HID), bspec(NE3, HID),
        wspec(NE3, HID, O2), bspec(NE3, O2),
    ]

    args = [
        x_t,
        c1_w0, c1_b0.reshape(H, HID, 1),
        c1_w1, c1_b1.reshape(H, HID, 1),
        c1_w2, c1_b2.reshape(H, O1, 1),
        c2_w0, c2_b0, c2_w1, c2_b1, c2_w2, c2_b2,
        c3_w0, c3_b0, c3_w1, c3_b1, c3_w2, c3_b2,
    ]

    out = pl.pallas_call(
        _line_kernel,
        grid=(H // LPB,),
        in_specs=in_specs,
        out_specs=pl.BlockSpec((LPB, 1, W), lambda h: (h, 0, 0)),
        out_shape=jax.ShapeDtypeStruct((H, 1, W), jnp.int32),
        compiler_params=pltpu.CompilerParams(
            dimension_semantics=("arbitrary",),
        ),
    )(*args)

    return out.reshape(1, 1, H, W)
